# baseline (speedup 1.0000x reference)
import jax
import jax.numpy as jnp
from jax import lax
from jax.experimental import pallas as pl
from jax.experimental.pallas import tpu as pltpu
from jax.experimental.pallas import tpu_sc as plsc

D = 1001
DP = 1008
NC, NS = 2, 16
NW = NC * NS
N = 4096 * 50
B_PER_W = N // NW
CH = 64
NCHUNK = B_PER_W // CH


def _exp_body(w_ref, o_ref):
    o_ref[...] = jnp.exp(w_ref[...])


_exp_table = pl.pallas_call(
    _exp_body,
    out_shape=jax.ShapeDtypeStruct((D, DP), jnp.float32),
)


def _gather_body(table_hbm, idx_hbm, out_hbm, idx_v, rows_v, pack_v, sem):
    wid = lax.axis_index("s") * NC + lax.axis_index("c")
    base = wid * B_PER_W

    def chunk(c, carry):
        off = base + c * CH
        pltpu.sync_copy(idx_hbm.at[pl.ds(off, CH)], idx_v)
        pltpu.async_copy(table_hbm.at[idx_v], rows_v, sem).wait()

        def row(r, carry2):
            dst = r * D
            for j in range(62):
                pack_v[pl.ds(dst + 16 * j, 16)] = rows_v[r, pl.ds(16 * j, 16)]
            pack_v[pl.ds(dst + (D - 16), 16)] = rows_v[r, pl.ds(D - 16, 16)]
            return carry2

        lax.fori_loop(0, CH, row, 0)
        pltpu.sync_copy(pack_v, out_hbm.at[pl.ds(off * D, CH * D)])
        return carry

    lax.fori_loop(0, NCHUNK, chunk, 0)


_gather = pl.kernel(
    _gather_body,
    out_type=jax.ShapeDtypeStruct((N * D,), jnp.float32),
    mesh=plsc.VectorSubcoreMesh(
        core_axis_name="c", subcore_axis_name="s", num_cores=NC, num_subcores=NS
    ),
    scratch_types=[
        pltpu.VMEM((CH,), jnp.int32),
        pltpu.VMEM((CH, DP), jnp.float32),
        pltpu.VMEM((CH * D,), jnp.float32),
        pltpu.SemaphoreType.DMA,
    ],
    compiler_params=pltpu.CompilerParams(use_tc_tiling_on_sc=False),
)


@jax.jit
def kernel(events, log_sigma_weight):
    w_pad = jnp.pad(log_sigma_weight, ((0, 0), (0, DP - D)))
    exp_table = _exp_table(w_pad)
    idx = events.reshape(N)
    out = _gather(exp_table, idx)
    return out.reshape(events.shape[0], events.shape[1], D)


# speedup vs baseline: 1.2459x; 1.2459x over previous
import jax
import jax.numpy as jnp
from jax import lax
from jax.experimental import pallas as pl
from jax.experimental.pallas import tpu as pltpu
from jax.experimental.pallas import tpu_sc as plsc

D = 1001
DP = 1008
NC, NS = 2, 16
NW = NC * NS
N = 4096 * 50
B_PER_W = N // NW
CH = 64
NCHUNK = B_PER_W // CH


def _exp_body(w_ref, o_ref):
    o_ref[...] = jnp.exp(w_ref[...])


_exp_table = pl.pallas_call(
    _exp_body,
    out_shape=jax.ShapeDtypeStruct((D, DP), jnp.float32),
)


def _gather_body(table_hbm, idx_hbm, out_hbm, idx_v, rows_v, pack_v, sem):
    wid = lax.axis_index("s") * NC + lax.axis_index("c")
    base = wid * B_PER_W

    def chunk(c, carry):
        off = base + c * CH
        pltpu.sync_copy(idx_hbm.at[pl.ds(off, CH)], idx_v)
        pltpu.async_copy(table_hbm.at[idx_v], rows_v, sem).wait()

        @plsc.parallel_loop(0, CH, step=1, unroll=2)
        def _row(r):
            dst = r * D
            for j in range(62):
                pack_v[pl.ds(dst + 16 * j, 16)] = rows_v[r, pl.ds(16 * j, 16)]
            pack_v[pl.ds(dst + (D - 16), 16)] = rows_v[r, pl.ds(D - 16, 16)]
        pltpu.sync_copy(pack_v, out_hbm.at[pl.ds(off * D, CH * D)])
        return carry

    lax.fori_loop(0, NCHUNK, chunk, 0)


_gather = pl.kernel(
    _gather_body,
    out_type=jax.ShapeDtypeStruct((N * D,), jnp.float32),
    mesh=plsc.VectorSubcoreMesh(
        core_axis_name="c", subcore_axis_name="s", num_cores=NC, num_subcores=NS
    ),
    scratch_types=[
        pltpu.VMEM((CH,), jnp.int32),
        pltpu.VMEM((CH, DP), jnp.float32),
        pltpu.VMEM((CH * D,), jnp.float32),
        pltpu.SemaphoreType.DMA,
    ],
    compiler_params=pltpu.CompilerParams(use_tc_tiling_on_sc=False),
)


@jax.jit
def kernel(events, log_sigma_weight):
    w_pad = jnp.pad(log_sigma_weight, ((0, 0), (0, DP - D)))
    exp_table = _exp_table(w_pad)
    idx = events.reshape(N)
    out = _gather(exp_table, idx)
    return out.reshape(events.shape[0], events.shape[1], D)


# speedup vs baseline: 3.0163x; 2.4210x over previous
import jax
import jax.numpy as jnp
from jax import lax
from jax.experimental import pallas as pl
from jax.experimental.pallas import tpu as pltpu
from jax.experimental.pallas import tpu_sc as plsc

D = 1001
DP = 1024
NC, NS = 2, 16
NW = NC * NS
N = 4096 * 50
B_PER_W = N // NW
CH = 64
NCHUNK = B_PER_W // CH


def _exp_body(w_ref, o_ref):
    o_ref[...] = jnp.exp(w_ref[...])


_exp_table = pl.pallas_call(
    _exp_body,
    out_shape=jax.ShapeDtypeStruct((D, DP), jnp.float32),
)


def _gather_body(table_hbm, idx_hbm, out_hbm, idx_all, rows_v, sem):
    wid = lax.axis_index("s") * NC + lax.axis_index("c")
    base = wid * B_PER_W
    pltpu.sync_copy(idx_hbm.at[pl.ds(base, B_PER_W)], idx_all)

    def chunk(c, carry):
        off = base + c * CH
        idx_c = idx_all.at[pl.ds(c * CH, CH)]
        pltpu.async_copy(table_hbm.at[idx_c], rows_v, sem).wait()
        pltpu.sync_copy(rows_v, out_hbm.at[pl.ds(off, CH)])
        return carry

    lax.fori_loop(0, NCHUNK, chunk, 0)


_gather = pl.kernel(
    _gather_body,
    out_type=jax.ShapeDtypeStruct((N, DP), jnp.float32),
    mesh=plsc.VectorSubcoreMesh(
        core_axis_name="c", subcore_axis_name="s", num_cores=NC, num_subcores=NS
    ),
    scratch_types=[
        pltpu.VMEM((B_PER_W,), jnp.int32),
        pltpu.VMEM((CH, DP), jnp.float32),
        pltpu.SemaphoreType.DMA,
    ],
    compiler_params=pltpu.CompilerParams(use_tc_tiling_on_sc=True),
)


@jax.jit
def kernel(events, log_sigma_weight):
    w_pad = jnp.pad(log_sigma_weight, ((0, 0), (0, DP - D)))
    exp_table = _exp_table(w_pad)
    idx = events.reshape(N)
    out = _gather(exp_table, idx)
    out = out[:, :D]
    return out.reshape(events.shape[0], events.shape[1], D)
